# baseline (speedup 1.0000x reference)
import functools

import jax
import jax.numpy as jnp
from jax import lax
from jax.experimental import pallas as pl
from jax.experimental.pallas import tpu as pltpu
from jax.experimental.pallas import tpu_sc as plsc

N = 100000
C = 128
B = 64
LANES = 16
NWORKERS = 32
ROWS_PER_W = 3128
CHUNK = 384
NCH = (ROWS_PER_W + CHUNK - 1) // CHUNK

TC_BLOCK = 4000


def _sc_stats_body(x_hbm, ids_hbm, sums_hbm, sqs_hbm, cnts_hbm,
                   xbuf0, xbuf1, ids_v, sum_acc, sq_acc, cnt_acc, seg_smem,
                   sem0, sem1):
    nc = 2
    wid = lax.axis_index("s") * nc + lax.axis_index("c")

    zeros16 = jnp.zeros((LANES,), jnp.float32)
    nvec = C // LANES

    def zero_body(b, _):
        for j in range(nvec):
            sum_acc[b, pl.ds(j * LANES, LANES)] = zeros16
            sq_acc[b, pl.ds(j * LANES, LANES)] = zeros16
        cnt_acc[b, :] = zeros16
        return 0

    lax.fori_loop(0, B, zero_body, 0)

    t0 = wid * ROWS_PER_W
    rows = lax.select(wid == NWORKERS - 1, N - (NWORKERS - 1) * ROWS_PER_W,
                      ROWS_PER_W)
    t1 = t0 + rows

    ids_start = jnp.minimum(t0, N - ROWS_PER_W)
    base_off = t0 - ids_start
    pltpu.sync_copy(ids_hbm.at[pl.ds(ids_start, ROWS_PER_W)],
                    ids_v.at[pl.ds(0, ROWS_PER_W)])

    b_lo = ids_v[pl.ds(base_off, LANES)][0]
    b_hi = ids_v[pl.ds(base_off + rows - 1, LANES)][0]

    def search_body(b, _):
        def step(_, lohi):
            slo, shi = lohi
            mid = (slo + shi) // 2
            v = ids_v[pl.ds(base_off + mid, LANES)][0]
            pred = v < b
            return (jnp.where(pred, mid + 1, slo), jnp.where(pred, shi, mid))

        slo, _shi = lax.fori_loop(0, 13, step, (0, rows))
        seg_smem[b] = slo
        return 0

    lax.fori_loop(b_lo, b_hi + 2, search_body, 0)

    bufs = (xbuf0, xbuf1)
    sems = (sem0, sem1)

    def start_dma(k):
        s = t0 + k * CHUNK
        sc = jnp.minimum(s, t1 - CHUNK)
        return pltpu.async_copy(x_hbm.at[pl.ds(sc, CHUNK), :],
                                bufs[k % 2], sems[k % 2])

    def process(k):
        s = t0 + k * CHUNK
        sc = jnp.minimum(s, t1 - CHUNK)
        e = jnp.minimum(s + CHUNK, t1)
        xbuf = bufs[k % 2]

        def seg_body(b, _):
            gs = t0 + seg_smem[b]
            ge = t0 + seg_smem[b + 1]
            r0 = jnp.maximum(gs, s)
            r1 = jnp.minimum(ge, e)

            def row_body(r, carry):
                new = []
                for j in range(nvec):
                    v = xbuf[r, pl.ds(j * LANES, LANES)]
                    new.append(carry[j] + v)
                for j in range(nvec):
                    v = xbuf[r, pl.ds(j * LANES, LANES)]
                    new.append(carry[nvec + j] + v * v)
                return tuple(new)

            init = tuple(zeros16 for _ in range(2 * nvec))
            acc = lax.fori_loop(r0 - sc, r1 - sc, row_body, init)

            @pl.when(r1 > r0)
            def _flush():
                for j in range(nvec):
                    plsc.addupdate(sum_acc.at[b, pl.ds(j * LANES, LANES)],
                                   acc[j])
                    plsc.addupdate(sq_acc.at[b, pl.ds(j * LANES, LANES)],
                                   acc[nvec + j])
                cnt = (r1 - r0).astype(jnp.float32)
                plsc.addupdate(cnt_acc.at[b],
                               lax.broadcast_in_dim(cnt, (LANES,), ()))

            return 0

        lax.fori_loop(b_lo, b_hi + 1, seg_body, 0)

    pending = start_dma(0)
    for k in range(NCH):
        nxt = start_dma(k + 1) if k + 1 < NCH else None
        pending.wait()
        process(k)
        pending = nxt

    pltpu.sync_copy(sum_acc, sums_hbm.at[wid])
    pltpu.sync_copy(sq_acc, sqs_hbm.at[wid])
    pltpu.sync_copy(cnt_acc, cnts_hbm.at[wid])


_sc_stats = pl.kernel(
    _sc_stats_body,
    out_type=(
        jax.ShapeDtypeStruct((NWORKERS, B, C), jnp.float32),
        jax.ShapeDtypeStruct((NWORKERS, B, C), jnp.float32),
        jax.ShapeDtypeStruct((NWORKERS, B, LANES), jnp.float32),
    ),
    mesh=plsc.VectorSubcoreMesh(core_axis_name="c", subcore_axis_name="s"),
    scratch_types=[
        pltpu.VMEM((CHUNK, C), jnp.float32),
        pltpu.VMEM((CHUNK, C), jnp.float32),
        pltpu.VMEM((ROWS_PER_W + LANES,), jnp.int32),
        pltpu.VMEM((B, C), jnp.float32),
        pltpu.VMEM((B, C), jnp.float32),
        pltpu.VMEM((B, LANES), jnp.float32),
        pltpu.SMEM((B + 8,), jnp.int32),
        pltpu.SemaphoreType.DMA,
        pltpu.SemaphoreType.DMA,
    ],
)


def _tc_norm_body(x_ref, sums_ref, sqs_ref, cnts_ref, out_ref,
                  tab_ref, offs_ref):
    i = pl.program_id(0)

    @pl.when(i == 0)
    def _finalize():
        total = jnp.sum(sums_ref[...], axis=0)
        sq_total = jnp.sum(sqs_ref[...], axis=0)
        cnt_raw = jnp.sum(cnts_ref[...], axis=0)[:, 0:1]
        cnt = jnp.maximum(cnt_raw, 1.0)
        mean = total / cnt
        var = jnp.maximum(sq_total / cnt - mean * mean, 0.0)
        rstd = lax.rsqrt(var + 1e-5)
        tab = jnp.concatenate([rstd, -mean * rstd], axis=1)
        tab_ref[...] = tab.astype(jnp.bfloat16)
        eye = (lax.broadcasted_iota(jnp.int32, (B, B), 0)
               == lax.broadcasted_iota(jnp.int32, (B, B), 1)).astype(jnp.float32)
        cnt_row = lax.dot_general(cnt_raw, eye, (((0,), (0,)), ((), ())),
                                  precision=lax.Precision.HIGHEST,
                                  preferred_element_type=jnp.float32)
        triu = (lax.broadcasted_iota(jnp.int32, (B, B), 0)
                < lax.broadcasted_iota(jnp.int32, (B, B), 1)).astype(jnp.float32)
        off_row = lax.dot_general(cnt_row, triu, (((1,), (0,)), ((), ())),
                                  precision=lax.Precision.HIGHEST,
                                  preferred_element_type=jnp.float32)
        offs_ref[0:1, :] = off_row
        offs_ref[1:2, :] = off_row + cnt_row

    gi = (lax.broadcasted_iota(jnp.int32, (TC_BLOCK, B), 0)
          + i * TC_BLOCK).astype(jnp.float32)
    onehot = ((gi >= offs_ref[0:1, :])
              & (gi < offs_ref[1:2, :])).astype(jnp.bfloat16)
    rows = lax.dot_general(
        onehot, tab_ref[...], (((1,), (0,)), ((), ())),
        preferred_element_type=jnp.float32)
    out_ref[...] = x_ref[...] * rows[:, :C] + rows[:, C:]


def _tc_normalize(x, sums, sqs, cnts):
    nblocks = N // TC_BLOCK
    return pl.pallas_call(
        _tc_norm_body,
        grid=(nblocks,),
        in_specs=[
            pl.BlockSpec((TC_BLOCK, C), lambda i: (i, 0)),
            pl.BlockSpec((NWORKERS, B, C), lambda i: (0, 0, 0)),
            pl.BlockSpec((NWORKERS, B, C), lambda i: (0, 0, 0)),
            pl.BlockSpec((NWORKERS, B, LANES), lambda i: (0, 0, 0)),
        ],
        out_specs=pl.BlockSpec((TC_BLOCK, C), lambda i: (i, 0)),
        out_shape=jax.ShapeDtypeStruct((N, C), jnp.float32),
        scratch_shapes=[
            pltpu.VMEM((B, 2 * C), jnp.bfloat16),
            pltpu.VMEM((2, B), jnp.float32),
        ],
    )(x, sums, sqs, cnts)


@jax.jit
def kernel(x, batch):
    ids = batch.astype(jnp.int32)
    sums, sqs, cnts = _sc_stats(x, ids)
    return _tc_normalize(x, sums, sqs, cnts)


# speedup vs baseline: 1.0318x; 1.0318x over previous
import functools

import jax
import jax.numpy as jnp
from jax import lax
from jax.experimental import pallas as pl
from jax.experimental.pallas import tpu as pltpu
from jax.experimental.pallas import tpu_sc as plsc

N = 100000
C = 128
B = 64
LANES = 16
NWORKERS = 32
ROWS_PER_W = 3128
CHUNK = 384
NCH = (ROWS_PER_W + CHUNK - 1) // CHUNK

TC_BLOCK = 5000


def _sc_stats_body(x_hbm, ids_hbm, sums_hbm, sqs_hbm, cnts_hbm,
                   xbuf0, xbuf1, ids_v, sum_acc, sq_acc, cnt_acc, seg_smem,
                   sem0, sem1):
    nc = 2
    wid = lax.axis_index("s") * nc + lax.axis_index("c")

    zeros16 = jnp.zeros((LANES,), jnp.float32)
    nvec = C // LANES

    def zero_body(b, _):
        for j in range(nvec):
            sum_acc[b, pl.ds(j * LANES, LANES)] = zeros16
            sq_acc[b, pl.ds(j * LANES, LANES)] = zeros16
        cnt_acc[b, :] = zeros16
        return 0

    lax.fori_loop(0, B, zero_body, 0)

    t0 = wid * ROWS_PER_W
    rows = lax.select(wid == NWORKERS - 1, N - (NWORKERS - 1) * ROWS_PER_W,
                      ROWS_PER_W)
    t1 = t0 + rows

    ids_start = jnp.minimum(t0, N - ROWS_PER_W)
    base_off = t0 - ids_start
    pltpu.sync_copy(ids_hbm.at[pl.ds(ids_start, ROWS_PER_W)],
                    ids_v.at[pl.ds(0, ROWS_PER_W)])

    b_lo = ids_v[pl.ds(base_off, LANES)][0]
    b_hi = ids_v[pl.ds(base_off + rows - 1, LANES)][0]

    def search_body(b, _):
        def step(_, lohi):
            slo, shi = lohi
            mid = (slo + shi) // 2
            v = ids_v[pl.ds(base_off + mid, LANES)][0]
            pred = v < b
            return (jnp.where(pred, mid + 1, slo), jnp.where(pred, shi, mid))

        slo, _shi = lax.fori_loop(0, 13, step, (0, rows))
        seg_smem[b] = slo
        return 0

    lax.fori_loop(b_lo, b_hi + 2, search_body, 0)

    bufs = (xbuf0, xbuf1)
    sems = (sem0, sem1)

    def start_dma(k):
        s = t0 + k * CHUNK
        sc = jnp.minimum(s, t1 - CHUNK)
        return pltpu.async_copy(x_hbm.at[pl.ds(sc, CHUNK), :],
                                bufs[k % 2], sems[k % 2])

    def process(k):
        s = t0 + k * CHUNK
        sc = jnp.minimum(s, t1 - CHUNK)
        e = jnp.minimum(s + CHUNK, t1)
        xbuf = bufs[k % 2]

        def seg_body(b, _):
            gs = t0 + seg_smem[b]
            ge = t0 + seg_smem[b + 1]
            r0 = jnp.maximum(gs, s)
            r1 = jnp.minimum(ge, e)

            def row_body(r, carry):
                new = []
                for j in range(nvec):
                    v = xbuf[r, pl.ds(j * LANES, LANES)]
                    new.append(carry[j] + v)
                for j in range(nvec):
                    v = xbuf[r, pl.ds(j * LANES, LANES)]
                    new.append(carry[nvec + j] + v * v)
                return tuple(new)

            init = tuple(zeros16 for _ in range(2 * nvec))
            acc = lax.fori_loop(r0 - sc, r1 - sc, row_body, init)

            @pl.when(r1 > r0)
            def _flush():
                for j in range(nvec):
                    plsc.addupdate(sum_acc.at[b, pl.ds(j * LANES, LANES)],
                                   acc[j])
                    plsc.addupdate(sq_acc.at[b, pl.ds(j * LANES, LANES)],
                                   acc[nvec + j])
                cnt = (r1 - r0).astype(jnp.float32)
                plsc.addupdate(cnt_acc.at[b],
                               lax.broadcast_in_dim(cnt, (LANES,), ()))

            return 0

        lax.fori_loop(b_lo, b_hi + 1, seg_body, 0)

    pending = start_dma(0)
    for k in range(NCH):
        nxt = start_dma(k + 1) if k + 1 < NCH else None
        pending.wait()
        process(k)
        pending = nxt

    pltpu.sync_copy(sum_acc, sums_hbm.at[wid])
    pltpu.sync_copy(sq_acc, sqs_hbm.at[wid])
    pltpu.sync_copy(cnt_acc, cnts_hbm.at[wid])


_sc_stats = pl.kernel(
    _sc_stats_body,
    out_type=(
        jax.ShapeDtypeStruct((NWORKERS, B, C), jnp.float32),
        jax.ShapeDtypeStruct((NWORKERS, B, C), jnp.float32),
        jax.ShapeDtypeStruct((NWORKERS, B, LANES), jnp.float32),
    ),
    mesh=plsc.VectorSubcoreMesh(core_axis_name="c", subcore_axis_name="s"),
    scratch_types=[
        pltpu.VMEM((CHUNK, C), jnp.float32),
        pltpu.VMEM((CHUNK, C), jnp.float32),
        pltpu.VMEM((ROWS_PER_W + LANES,), jnp.int32),
        pltpu.VMEM((B, C), jnp.float32),
        pltpu.VMEM((B, C), jnp.float32),
        pltpu.VMEM((B, LANES), jnp.float32),
        pltpu.SMEM((B + 8,), jnp.int32),
        pltpu.SemaphoreType.DMA,
        pltpu.SemaphoreType.DMA,
    ],
)


def _tc_norm_body(x_ref, sums_ref, sqs_ref, cnts_ref, out_ref,
                  tab_ref, offs_ref):
    i = pl.program_id(0)

    @pl.when(i == 0)
    def _finalize():
        total = jnp.sum(sums_ref[...], axis=0)
        sq_total = jnp.sum(sqs_ref[...], axis=0)
        cnt_raw = jnp.sum(cnts_ref[...], axis=0)[:, 0:1]
        cnt = jnp.maximum(cnt_raw, 1.0)
        mean = total / cnt
        var = jnp.maximum(sq_total / cnt - mean * mean, 0.0)
        rstd = lax.rsqrt(var + 1e-5)
        tab = jnp.concatenate([rstd, -mean * rstd], axis=1)
        tab_ref[...] = tab.astype(jnp.bfloat16)
        eye = (lax.broadcasted_iota(jnp.int32, (B, B), 0)
               == lax.broadcasted_iota(jnp.int32, (B, B), 1)).astype(jnp.float32)
        cnt_row = lax.dot_general(cnt_raw, eye, (((0,), (0,)), ((), ())),
                                  precision=lax.Precision.HIGHEST,
                                  preferred_element_type=jnp.float32)
        triu = (lax.broadcasted_iota(jnp.int32, (B, B), 0)
                < lax.broadcasted_iota(jnp.int32, (B, B), 1)).astype(jnp.float32)
        off_row = lax.dot_general(cnt_row, triu, (((1,), (0,)), ((), ())),
                                  precision=lax.Precision.HIGHEST,
                                  preferred_element_type=jnp.float32)
        offs_ref[0:1, :] = off_row
        offs_ref[1:2, :] = off_row + cnt_row

    gi = (lax.broadcasted_iota(jnp.int32, (TC_BLOCK, B), 0)
          + i * TC_BLOCK).astype(jnp.float32)
    onehot = ((gi >= offs_ref[0:1, :])
              & (gi < offs_ref[1:2, :])).astype(jnp.bfloat16)
    rows = lax.dot_general(
        onehot, tab_ref[...], (((1,), (0,)), ((), ())),
        preferred_element_type=jnp.float32)
    out_ref[...] = x_ref[...] * rows[:, :C] + rows[:, C:]


def _tc_normalize(x, sums, sqs, cnts):
    nblocks = N // TC_BLOCK
    return pl.pallas_call(
        _tc_norm_body,
        grid=(nblocks,),
        in_specs=[
            pl.BlockSpec((TC_BLOCK, C), lambda i: (i, 0)),
            pl.BlockSpec((NWORKERS, B, C), lambda i: (0, 0, 0)),
            pl.BlockSpec((NWORKERS, B, C), lambda i: (0, 0, 0)),
            pl.BlockSpec((NWORKERS, B, LANES), lambda i: (0, 0, 0)),
        ],
        out_specs=pl.BlockSpec((TC_BLOCK, C), lambda i: (i, 0)),
        out_shape=jax.ShapeDtypeStruct((N, C), jnp.float32),
        scratch_shapes=[
            pltpu.VMEM((B, 2 * C), jnp.bfloat16),
            pltpu.VMEM((2, B), jnp.float32),
        ],
    )(x, sums, sqs, cnts)


@jax.jit
def kernel(x, batch):
    ids = batch.astype(jnp.int32)
    sums, sqs, cnts = _sc_stats(x, ids)
    return _tc_normalize(x, sums, sqs, cnts)


# speedup vs baseline: 1.0819x; 1.0486x over previous
import functools

import jax
import jax.numpy as jnp
from jax import lax
from jax.experimental import pallas as pl
from jax.experimental.pallas import tpu as pltpu
from jax.experimental.pallas import tpu_sc as plsc

N = 100000
C = 128
B = 64
LANES = 16
NWORKERS = 32
ROWS_PER_W = 3128
CHUNK = 384
NCH = (ROWS_PER_W + CHUNK - 1) // CHUNK

TC_BLOCK = 10000


def _sc_stats_body(x_hbm, ids_hbm, sums_hbm, sqs_hbm, cnts_hbm,
                   xbuf0, xbuf1, ids_v, sum_acc, sq_acc, cnt_acc, seg_smem,
                   sem0, sem1):
    nc = 2
    wid = lax.axis_index("s") * nc + lax.axis_index("c")

    zeros16 = jnp.zeros((LANES,), jnp.float32)
    nvec = C // LANES

    def zero_body(b, _):
        for j in range(nvec):
            sum_acc[b, pl.ds(j * LANES, LANES)] = zeros16
            sq_acc[b, pl.ds(j * LANES, LANES)] = zeros16
        cnt_acc[b, :] = zeros16
        return 0

    lax.fori_loop(0, B, zero_body, 0)

    t0 = wid * ROWS_PER_W
    rows = lax.select(wid == NWORKERS - 1, N - (NWORKERS - 1) * ROWS_PER_W,
                      ROWS_PER_W)
    t1 = t0 + rows

    ids_start = jnp.minimum(t0, N - ROWS_PER_W)
    base_off = t0 - ids_start
    pltpu.sync_copy(ids_hbm.at[pl.ds(ids_start, ROWS_PER_W)],
                    ids_v.at[pl.ds(0, ROWS_PER_W)])

    b_lo = ids_v[pl.ds(base_off, LANES)][0]
    b_hi = ids_v[pl.ds(base_off + rows - 1, LANES)][0]

    def search_body(b, _):
        def step(_, lohi):
            slo, shi = lohi
            mid = (slo + shi) // 2
            v = ids_v[pl.ds(base_off + mid, LANES)][0]
            pred = v < b
            return (jnp.where(pred, mid + 1, slo), jnp.where(pred, shi, mid))

        slo, _shi = lax.fori_loop(0, 13, step, (0, rows))
        seg_smem[b] = slo
        return 0

    lax.fori_loop(b_lo, b_hi + 2, search_body, 0)

    bufs = (xbuf0, xbuf1)
    sems = (sem0, sem1)

    def start_dma(k):
        s = t0 + k * CHUNK
        sc = jnp.minimum(s, t1 - CHUNK)
        return pltpu.async_copy(x_hbm.at[pl.ds(sc, CHUNK), :],
                                bufs[k % 2], sems[k % 2])

    def process(k):
        s = t0 + k * CHUNK
        sc = jnp.minimum(s, t1 - CHUNK)
        e = jnp.minimum(s + CHUNK, t1)
        xbuf = bufs[k % 2]

        def seg_body(b, _):
            gs = t0 + seg_smem[b]
            ge = t0 + seg_smem[b + 1]
            r0 = jnp.maximum(gs, s)
            r1 = jnp.minimum(ge, e)

            def row_body(r, carry):
                new = []
                for j in range(nvec):
                    v = xbuf[r, pl.ds(j * LANES, LANES)]
                    new.append(carry[j] + v)
                for j in range(nvec):
                    v = xbuf[r, pl.ds(j * LANES, LANES)]
                    new.append(carry[nvec + j] + v * v)
                return tuple(new)

            init = tuple(zeros16 for _ in range(2 * nvec))
            acc = lax.fori_loop(r0 - sc, r1 - sc, row_body, init)

            @pl.when(r1 > r0)
            def _flush():
                for j in range(nvec):
                    plsc.addupdate(sum_acc.at[b, pl.ds(j * LANES, LANES)],
                                   acc[j])
                    plsc.addupdate(sq_acc.at[b, pl.ds(j * LANES, LANES)],
                                   acc[nvec + j])
                cnt = (r1 - r0).astype(jnp.float32)
                plsc.addupdate(cnt_acc.at[b],
                               lax.broadcast_in_dim(cnt, (LANES,), ()))

            return 0

        lax.fori_loop(b_lo, b_hi + 1, seg_body, 0)

    pending = start_dma(0)
    for k in range(NCH):
        nxt = start_dma(k + 1) if k + 1 < NCH else None
        pending.wait()
        process(k)
        pending = nxt

    pltpu.sync_copy(sum_acc, sums_hbm.at[wid])
    pltpu.sync_copy(sq_acc, sqs_hbm.at[wid])
    pltpu.sync_copy(cnt_acc, cnts_hbm.at[wid])


_sc_stats = pl.kernel(
    _sc_stats_body,
    out_type=(
        jax.ShapeDtypeStruct((NWORKERS, B, C), jnp.float32),
        jax.ShapeDtypeStruct((NWORKERS, B, C), jnp.float32),
        jax.ShapeDtypeStruct((NWORKERS, B, LANES), jnp.float32),
    ),
    mesh=plsc.VectorSubcoreMesh(core_axis_name="c", subcore_axis_name="s"),
    scratch_types=[
        pltpu.VMEM((CHUNK, C), jnp.float32),
        pltpu.VMEM((CHUNK, C), jnp.float32),
        pltpu.VMEM((ROWS_PER_W + LANES,), jnp.int32),
        pltpu.VMEM((B, C), jnp.float32),
        pltpu.VMEM((B, C), jnp.float32),
        pltpu.VMEM((B, LANES), jnp.float32),
        pltpu.SMEM((B + 8,), jnp.int32),
        pltpu.SemaphoreType.DMA,
        pltpu.SemaphoreType.DMA,
    ],
)


def _tc_norm_body(x_ref, sums_ref, sqs_ref, cnts_ref, out_ref,
                  tab_ref, offs_ref):
    i = pl.program_id(0)

    @pl.when(i == 0)
    def _finalize():
        total = jnp.sum(sums_ref[...], axis=0)
        sq_total = jnp.sum(sqs_ref[...], axis=0)
        cnt_raw = jnp.sum(cnts_ref[...], axis=0)[:, 0:1]
        cnt = jnp.maximum(cnt_raw, 1.0)
        mean = total / cnt
        var = jnp.maximum(sq_total / cnt - mean * mean, 0.0)
        rstd = lax.rsqrt(var + 1e-5)
        tab = jnp.concatenate([rstd, -mean * rstd], axis=1)
        tab_ref[...] = tab.astype(jnp.bfloat16)
        eye = (lax.broadcasted_iota(jnp.int32, (B, B), 0)
               == lax.broadcasted_iota(jnp.int32, (B, B), 1)).astype(jnp.float32)
        cnt_row = lax.dot_general(cnt_raw, eye, (((0,), (0,)), ((), ())),
                                  precision=lax.Precision.HIGHEST,
                                  preferred_element_type=jnp.float32)
        triu = (lax.broadcasted_iota(jnp.int32, (B, B), 0)
                < lax.broadcasted_iota(jnp.int32, (B, B), 1)).astype(jnp.float32)
        off_row = lax.dot_general(cnt_row, triu, (((1,), (0,)), ((), ())),
                                  precision=lax.Precision.HIGHEST,
                                  preferred_element_type=jnp.float32)
        offs_ref[0:1, :] = off_row
        offs_ref[1:2, :] = off_row + cnt_row

    gi = (lax.broadcasted_iota(jnp.int32, (TC_BLOCK, B), 0)
          + i * TC_BLOCK).astype(jnp.float32)
    onehot = ((gi >= offs_ref[0:1, :])
              & (gi < offs_ref[1:2, :])).astype(jnp.bfloat16)
    rows = lax.dot_general(
        onehot, tab_ref[...], (((1,), (0,)), ((), ())),
        preferred_element_type=jnp.float32)
    out_ref[...] = x_ref[...] * rows[:, :C] + rows[:, C:]


def _tc_normalize(x, sums, sqs, cnts):
    nblocks = N // TC_BLOCK
    return pl.pallas_call(
        _tc_norm_body,
        grid=(nblocks,),
        in_specs=[
            pl.BlockSpec((TC_BLOCK, C), lambda i: (i, 0)),
            pl.BlockSpec((NWORKERS, B, C), lambda i: (0, 0, 0)),
            pl.BlockSpec((NWORKERS, B, C), lambda i: (0, 0, 0)),
            pl.BlockSpec((NWORKERS, B, LANES), lambda i: (0, 0, 0)),
        ],
        out_specs=pl.BlockSpec((TC_BLOCK, C), lambda i: (i, 0)),
        out_shape=jax.ShapeDtypeStruct((N, C), jnp.float32),
        scratch_shapes=[
            pltpu.VMEM((B, 2 * C), jnp.bfloat16),
            pltpu.VMEM((2, B), jnp.float32),
        ],
    )(x, sums, sqs, cnts)


@jax.jit
def kernel(x, batch):
    ids = batch.astype(jnp.int32)
    sums, sqs, cnts = _sc_stats(x, ids)
    return _tc_normalize(x, sums, sqs, cnts)


# speedup vs baseline: 1.0863x; 1.0041x over previous
import functools

import jax
import jax.numpy as jnp
from jax import lax
from jax.experimental import pallas as pl
from jax.experimental.pallas import tpu as pltpu
from jax.experimental.pallas import tpu_sc as plsc

N = 100000
C = 128
B = 64
LANES = 16
NWORKERS = 32
ROWS_PER_W = 3128
CHUNK = 384
NCH = (ROWS_PER_W + CHUNK - 1) // CHUNK

TC_BLOCK = 20000


def _sc_stats_body(x_hbm, ids_hbm, sums_hbm, sqs_hbm, cnts_hbm,
                   xbuf0, xbuf1, ids_v, sum_acc, sq_acc, cnt_acc, seg_smem,
                   sem0, sem1):
    nc = 2
    wid = lax.axis_index("s") * nc + lax.axis_index("c")

    zeros16 = jnp.zeros((LANES,), jnp.float32)
    nvec = C // LANES

    def zero_body(b, _):
        for j in range(nvec):
            sum_acc[b, pl.ds(j * LANES, LANES)] = zeros16
            sq_acc[b, pl.ds(j * LANES, LANES)] = zeros16
        cnt_acc[b, :] = zeros16
        return 0

    lax.fori_loop(0, B, zero_body, 0)

    t0 = wid * ROWS_PER_W
    rows = lax.select(wid == NWORKERS - 1, N - (NWORKERS - 1) * ROWS_PER_W,
                      ROWS_PER_W)
    t1 = t0 + rows

    ids_start = jnp.minimum(t0, N - ROWS_PER_W)
    base_off = t0 - ids_start
    pltpu.sync_copy(ids_hbm.at[pl.ds(ids_start, ROWS_PER_W)],
                    ids_v.at[pl.ds(0, ROWS_PER_W)])

    b_lo = ids_v[pl.ds(base_off, LANES)][0]
    b_hi = ids_v[pl.ds(base_off + rows - 1, LANES)][0]

    def search_body(b, _):
        def step(_, lohi):
            slo, shi = lohi
            mid = (slo + shi) // 2
            v = ids_v[pl.ds(base_off + mid, LANES)][0]
            pred = v < b
            return (jnp.where(pred, mid + 1, slo), jnp.where(pred, shi, mid))

        slo, _shi = lax.fori_loop(0, 13, step, (0, rows))
        seg_smem[b] = slo
        return 0

    lax.fori_loop(b_lo, b_hi + 2, search_body, 0)

    bufs = (xbuf0, xbuf1)
    sems = (sem0, sem1)

    def start_dma(k):
        s = t0 + k * CHUNK
        sc = jnp.minimum(s, t1 - CHUNK)
        return pltpu.async_copy(x_hbm.at[pl.ds(sc, CHUNK), :],
                                bufs[k % 2], sems[k % 2])

    def process(k):
        s = t0 + k * CHUNK
        sc = jnp.minimum(s, t1 - CHUNK)
        e = jnp.minimum(s + CHUNK, t1)
        xbuf = bufs[k % 2]

        def seg_body(b, _):
            gs = t0 + seg_smem[b]
            ge = t0 + seg_smem[b + 1]
            r0 = jnp.maximum(gs, s)
            r1 = jnp.minimum(ge, e)

            def row_body(r, carry):
                new = []
                for j in range(nvec):
                    v = xbuf[r, pl.ds(j * LANES, LANES)]
                    new.append(carry[j] + v)
                for j in range(nvec):
                    v = xbuf[r, pl.ds(j * LANES, LANES)]
                    new.append(carry[nvec + j] + v * v)
                return tuple(new)

            init = tuple(zeros16 for _ in range(2 * nvec))
            acc = lax.fori_loop(r0 - sc, r1 - sc, row_body, init)

            @pl.when(r1 > r0)
            def _flush():
                for j in range(nvec):
                    plsc.addupdate(sum_acc.at[b, pl.ds(j * LANES, LANES)],
                                   acc[j])
                    plsc.addupdate(sq_acc.at[b, pl.ds(j * LANES, LANES)],
                                   acc[nvec + j])
                cnt = (r1 - r0).astype(jnp.float32)
                plsc.addupdate(cnt_acc.at[b],
                               lax.broadcast_in_dim(cnt, (LANES,), ()))

            return 0

        lax.fori_loop(b_lo, b_hi + 1, seg_body, 0)

    pending = start_dma(0)
    for k in range(NCH):
        nxt = start_dma(k + 1) if k + 1 < NCH else None
        pending.wait()
        process(k)
        pending = nxt

    pltpu.sync_copy(sum_acc, sums_hbm.at[wid])
    pltpu.sync_copy(sq_acc, sqs_hbm.at[wid])
    pltpu.sync_copy(cnt_acc, cnts_hbm.at[wid])


_sc_stats = pl.kernel(
    _sc_stats_body,
    out_type=(
        jax.ShapeDtypeStruct((NWORKERS, B, C), jnp.float32),
        jax.ShapeDtypeStruct((NWORKERS, B, C), jnp.float32),
        jax.ShapeDtypeStruct((NWORKERS, B, LANES), jnp.float32),
    ),
    mesh=plsc.VectorSubcoreMesh(core_axis_name="c", subcore_axis_name="s"),
    scratch_types=[
        pltpu.VMEM((CHUNK, C), jnp.float32),
        pltpu.VMEM((CHUNK, C), jnp.float32),
        pltpu.VMEM((ROWS_PER_W + LANES,), jnp.int32),
        pltpu.VMEM((B, C), jnp.float32),
        pltpu.VMEM((B, C), jnp.float32),
        pltpu.VMEM((B, LANES), jnp.float32),
        pltpu.SMEM((B + 8,), jnp.int32),
        pltpu.SemaphoreType.DMA,
        pltpu.SemaphoreType.DMA,
    ],
)


def _tc_norm_body(x_ref, sums_ref, sqs_ref, cnts_ref, out_ref,
                  tab_ref, offs_ref):
    i = pl.program_id(0)

    @pl.when(i == 0)
    def _finalize():
        total = jnp.sum(sums_ref[...], axis=0)
        sq_total = jnp.sum(sqs_ref[...], axis=0)
        cnt_raw = jnp.sum(cnts_ref[...], axis=0)[:, 0:1]
        cnt = jnp.maximum(cnt_raw, 1.0)
        mean = total / cnt
        var = jnp.maximum(sq_total / cnt - mean * mean, 0.0)
        rstd = lax.rsqrt(var + 1e-5)
        tab = jnp.concatenate([rstd, -mean * rstd], axis=1)
        tab_ref[...] = tab.astype(jnp.bfloat16)
        eye = (lax.broadcasted_iota(jnp.int32, (B, B), 0)
               == lax.broadcasted_iota(jnp.int32, (B, B), 1)).astype(jnp.float32)
        cnt_row = lax.dot_general(cnt_raw, eye, (((0,), (0,)), ((), ())),
                                  precision=lax.Precision.HIGHEST,
                                  preferred_element_type=jnp.float32)
        triu = (lax.broadcasted_iota(jnp.int32, (B, B), 0)
                < lax.broadcasted_iota(jnp.int32, (B, B), 1)).astype(jnp.float32)
        off_row = lax.dot_general(cnt_row, triu, (((1,), (0,)), ((), ())),
                                  precision=lax.Precision.HIGHEST,
                                  preferred_element_type=jnp.float32)
        offs_ref[0:1, :] = off_row
        offs_ref[1:2, :] = off_row + cnt_row

    gi = (lax.broadcasted_iota(jnp.int32, (TC_BLOCK, B), 0)
          + i * TC_BLOCK).astype(jnp.float32)
    onehot = ((gi >= offs_ref[0:1, :])
              & (gi < offs_ref[1:2, :])).astype(jnp.bfloat16)
    rows = lax.dot_general(
        onehot, tab_ref[...], (((1,), (0,)), ((), ())),
        preferred_element_type=jnp.float32)
    out_ref[...] = x_ref[...] * rows[:, :C] + rows[:, C:]


def _tc_normalize(x, sums, sqs, cnts):
    nblocks = N // TC_BLOCK
    return pl.pallas_call(
        _tc_norm_body,
        grid=(nblocks,),
        in_specs=[
            pl.BlockSpec((TC_BLOCK, C), lambda i: (i, 0)),
            pl.BlockSpec((NWORKERS, B, C), lambda i: (0, 0, 0)),
            pl.BlockSpec((NWORKERS, B, C), lambda i: (0, 0, 0)),
            pl.BlockSpec((NWORKERS, B, LANES), lambda i: (0, 0, 0)),
        ],
        out_specs=pl.BlockSpec((TC_BLOCK, C), lambda i: (i, 0)),
        out_shape=jax.ShapeDtypeStruct((N, C), jnp.float32),
        scratch_shapes=[
            pltpu.VMEM((B, 2 * C), jnp.bfloat16),
            pltpu.VMEM((2, B), jnp.float32),
        ],
    )(x, sums, sqs, cnts)


@jax.jit
def kernel(x, batch):
    ids = batch.astype(jnp.int32)
    sums, sqs, cnts = _sc_stats(x, ids)
    return _tc_normalize(x, sums, sqs, cnts)


# speedup vs baseline: 1.1329x; 1.0429x over previous
import functools

import jax
import jax.numpy as jnp
from jax import lax
from jax.experimental import pallas as pl
from jax.experimental.pallas import tpu as pltpu
from jax.experimental.pallas import tpu_sc as plsc

N = 100000
C = 128
B = 64
LANES = 16
NWORKERS = 32
ROWS_PER_W = 3128
CHUNK = 384
NCH = (ROWS_PER_W + CHUNK - 1) // CHUNK

TC_BLOCK = 20000


def _sc_stats_body(x_hbm, ids_hbm, sums_hbm, sqs_hbm, cnts_hbm,
                   xbuf0, xbuf1, ids_v, sum_acc, sq_acc, cnt_acc, seg_smem,
                   sem0, sem1):
    nc = 2
    wid = lax.axis_index("s") * nc + lax.axis_index("c")

    zeros16 = jnp.zeros((LANES,), jnp.float32)
    nvec = C // LANES

    def zero_body(b, _):
        for j in range(nvec):
            sum_acc[b, pl.ds(j * LANES, LANES)] = zeros16
            sq_acc[b, pl.ds(j * LANES, LANES)] = zeros16
        cnt_acc[b, :] = zeros16
        return 0

    lax.fori_loop(0, B, zero_body, 0)

    t0 = wid * ROWS_PER_W
    rows = lax.select(wid == NWORKERS - 1, N - (NWORKERS - 1) * ROWS_PER_W,
                      ROWS_PER_W)
    t1 = t0 + rows

    bufs = (xbuf0, xbuf1)
    sems = (sem0, sem1)

    def start_dma(k):
        s = t0 + k * CHUNK
        sc = jnp.minimum(s, t1 - CHUNK)
        return pltpu.async_copy(x_hbm.at[pl.ds(sc, CHUNK), :],
                                bufs[k % 2], sems[k % 2])

    pending = start_dma(0)

    ids_start = jnp.minimum(t0, N - ROWS_PER_W)
    base_off = t0 - ids_start
    pltpu.sync_copy(ids_hbm.at[pl.ds(ids_start, ROWS_PER_W)],
                    ids_v.at[pl.ds(0, ROWS_PER_W)])

    b_lo = ids_v[pl.ds(base_off, LANES)][0]
    b_hi = ids_v[pl.ds(base_off + rows - 1, LANES)][0]

    def search_body(b, _):
        def step(_, lohi):
            slo, shi = lohi
            mid = (slo + shi) // 2
            v = ids_v[pl.ds(base_off + mid, LANES)][0]
            pred = v < b
            return (jnp.where(pred, mid + 1, slo), jnp.where(pred, shi, mid))

        slo, _shi = lax.fori_loop(0, 13, step, (0, rows))
        seg_smem[b] = slo
        return 0

    lax.fori_loop(b_lo, b_hi + 2, search_body, 0)

    def process(k):
        s = t0 + k * CHUNK
        sc = jnp.minimum(s, t1 - CHUNK)
        e = jnp.minimum(s + CHUNK, t1)
        xbuf = bufs[k % 2]

        def seg_body(b, _):
            gs = t0 + seg_smem[b]
            ge = t0 + seg_smem[b + 1]
            r0 = jnp.maximum(gs, s)
            r1 = jnp.minimum(ge, e)

            def row_body(r, carry):
                new = []
                for j in range(nvec):
                    v = xbuf[r, pl.ds(j * LANES, LANES)]
                    new.append(carry[j] + v)
                for j in range(nvec):
                    v = xbuf[r, pl.ds(j * LANES, LANES)]
                    new.append(carry[nvec + j] + v * v)
                return tuple(new)

            init = tuple(zeros16 for _ in range(2 * nvec))
            acc = lax.fori_loop(r0 - sc, r1 - sc, row_body, init)

            @pl.when(r1 > r0)
            def _flush():
                for j in range(nvec):
                    plsc.addupdate(sum_acc.at[b, pl.ds(j * LANES, LANES)],
                                   acc[j])
                    plsc.addupdate(sq_acc.at[b, pl.ds(j * LANES, LANES)],
                                   acc[nvec + j])
                cnt = (r1 - r0).astype(jnp.float32)
                plsc.addupdate(cnt_acc.at[b],
                               lax.broadcast_in_dim(cnt, (LANES,), ()))

            return 0

        lax.fori_loop(b_lo, b_hi + 1, seg_body, 0)

    for k in range(NCH):
        nxt = start_dma(k + 1) if k + 1 < NCH else None
        pending.wait()
        process(k)
        pending = nxt

    pltpu.sync_copy(sum_acc, sums_hbm.at[wid])
    pltpu.sync_copy(sq_acc, sqs_hbm.at[wid])
    pltpu.sync_copy(cnt_acc, cnts_hbm.at[wid])


_sc_stats = pl.kernel(
    _sc_stats_body,
    out_type=(
        jax.ShapeDtypeStruct((NWORKERS, B, C), jnp.float32),
        jax.ShapeDtypeStruct((NWORKERS, B, C), jnp.float32),
        jax.ShapeDtypeStruct((NWORKERS, B, LANES), jnp.float32),
    ),
    mesh=plsc.VectorSubcoreMesh(core_axis_name="c", subcore_axis_name="s"),
    scratch_types=[
        pltpu.VMEM((CHUNK, C), jnp.float32),
        pltpu.VMEM((CHUNK, C), jnp.float32),
        pltpu.VMEM((ROWS_PER_W + LANES,), jnp.int32),
        pltpu.VMEM((B, C), jnp.float32),
        pltpu.VMEM((B, C), jnp.float32),
        pltpu.VMEM((B, LANES), jnp.float32),
        pltpu.SMEM((B + 8,), jnp.int32),
        pltpu.SemaphoreType.DMA,
        pltpu.SemaphoreType.DMA,
    ],
)


def _tc_norm_body(x_ref, sums_ref, sqs_ref, cnts_ref, out_ref,
                  tab_ref, offs_ref):
    i = pl.program_id(0)

    @pl.when(i == 0)
    def _finalize():
        total = jnp.sum(sums_ref[...], axis=0)
        sq_total = jnp.sum(sqs_ref[...], axis=0)
        cnt_raw = jnp.sum(cnts_ref[...], axis=0)[:, 0:1]
        cnt = jnp.maximum(cnt_raw, 1.0)
        mean = total / cnt
        var = jnp.maximum(sq_total / cnt - mean * mean, 0.0)
        rstd = lax.rsqrt(var + 1e-5)
        tab = jnp.concatenate([rstd, -mean * rstd], axis=1)
        tab_ref[...] = tab.astype(jnp.bfloat16)
        eye = (lax.broadcasted_iota(jnp.int32, (B, B), 0)
               == lax.broadcasted_iota(jnp.int32, (B, B), 1)).astype(jnp.float32)
        cnt_row = lax.dot_general(cnt_raw, eye, (((0,), (0,)), ((), ())),
                                  precision=lax.Precision.HIGHEST,
                                  preferred_element_type=jnp.float32)
        triu = (lax.broadcasted_iota(jnp.int32, (B, B), 0)
                < lax.broadcasted_iota(jnp.int32, (B, B), 1)).astype(jnp.float32)
        off_row = lax.dot_general(cnt_row, triu, (((1,), (0,)), ((), ())),
                                  precision=lax.Precision.HIGHEST,
                                  preferred_element_type=jnp.float32)
        offs_ref[0:1, :] = off_row
        offs_ref[1:2, :] = off_row + cnt_row

    gi = (lax.broadcasted_iota(jnp.int32, (TC_BLOCK, B), 0)
          + i * TC_BLOCK).astype(jnp.float32)
    onehot = ((gi >= offs_ref[0:1, :])
              & (gi < offs_ref[1:2, :])).astype(jnp.bfloat16)
    rows = lax.dot_general(
        onehot, tab_ref[...], (((1,), (0,)), ((), ())),
        preferred_element_type=jnp.float32)
    out_ref[...] = x_ref[...] * rows[:, :C] + rows[:, C:]


def _tc_normalize(x, sums, sqs, cnts):
    nblocks = N // TC_BLOCK
    return pl.pallas_call(
        _tc_norm_body,
        grid=(nblocks,),
        in_specs=[
            pl.BlockSpec((TC_BLOCK, C), lambda i: (i, 0)),
            pl.BlockSpec((NWORKERS, B, C), lambda i: (0, 0, 0)),
            pl.BlockSpec((NWORKERS, B, C), lambda i: (0, 0, 0)),
            pl.BlockSpec((NWORKERS, B, LANES), lambda i: (0, 0, 0)),
        ],
        out_specs=pl.BlockSpec((TC_BLOCK, C), lambda i: (i, 0)),
        out_shape=jax.ShapeDtypeStruct((N, C), jnp.float32),
        scratch_shapes=[
            pltpu.VMEM((B, 2 * C), jnp.bfloat16),
            pltpu.VMEM((2, B), jnp.float32),
        ],
    )(x, sums, sqs, cnts)


@jax.jit
def kernel(x, batch):
    ids = batch.astype(jnp.int32)
    sums, sqs, cnts = _sc_stats(x, ids)
    return _tc_normalize(x, sums, sqs, cnts)
